# Initial kernel scaffold; baseline (speedup 1.0000x reference)
import jax
import jax.numpy as jnp
from jax.experimental import pallas as pl


def kernel(x, Wr, We):
    raise NotImplementedError("write your pallas kernel here")


# speedup vs baseline: 2.9261x; 2.9261x over previous
import jax
import jax.numpy as jnp
from jax.experimental import pallas as pl
from jax.experimental.pallas import tpu as pltpu


def _moe_block_kernel(x_ref, wr_ref, we_ref,
                      ns_ref, ti_ref, mask_ref, rp_ref, probs_ref):
    e = pl.program_id(1)
    num_e = pl.num_programs(1)
    x = x_ref[...]

    @pl.when(e == 0)
    def _router():
        logits = jnp.dot(x, wr_ref[...], preferred_element_type=jnp.float32)
        m = jnp.max(logits, axis=-1, keepdims=True)
        ex = jnp.exp(logits - m)
        probs = ex / jnp.sum(ex, axis=-1, keepdims=True)
        ti = jnp.argmax(probs, axis=-1)
        onehot = (jax.lax.broadcasted_iota(jnp.int32, probs.shape, 1)
                  == ti[:, None]).astype(jnp.float32)
        probs_ref[...] = probs
        mask_ref[...] = onehot
        rp_ref[...] = onehot
        ti_ref[...] = ti[:, None].astype(jnp.int32)
        ns_ref[...] = jnp.zeros_like(ns_ref)

    sel = (ti_ref[...] == e).astype(jnp.float32)
    mm = jnp.dot(x, we_ref[0], preferred_element_type=jnp.float32)
    ns_ref[...] += sel * mm


def kernel(x, Wr, We):
    input_shape = x.shape
    D = x.shape[-1]
    E = Wr.shape[-1]
    xf = x.reshape(-1, D)
    N = xf.shape[0]
    T = 2048
    num_t = N // T

    grid = (num_t, E)
    out_shapes = (
        jax.ShapeDtypeStruct((N, D), jnp.float32),
        jax.ShapeDtypeStruct((N, 1), jnp.int32),
        jax.ShapeDtypeStruct((N, E), jnp.float32),
        jax.ShapeDtypeStruct((N, E), jnp.float32),
        jax.ShapeDtypeStruct((N, E), jnp.float32),
    )
    ns, ti, mask, rp, probs = pl.pallas_call(
        _moe_block_kernel,
        grid=grid,
        in_specs=[
            pl.BlockSpec((T, D), lambda t, e: (t, 0)),
            pl.BlockSpec((D, E), lambda t, e: (0, 0)),
            pl.BlockSpec((1, D, D), lambda t, e: (e, 0, 0)),
        ],
        out_specs=(
            pl.BlockSpec((T, D), lambda t, e: (t, 0)),
            pl.BlockSpec((T, 1), lambda t, e: (t, 0)),
            pl.BlockSpec((T, E), lambda t, e: (t, 0)),
            pl.BlockSpec((T, E), lambda t, e: (t, 0)),
            pl.BlockSpec((T, E), lambda t, e: (t, 0)),
        ),
        out_shape=out_shapes,
        compiler_params=pltpu.CompilerParams(
            dimension_semantics=("parallel", "arbitrary"),
        ),
    )(xf, Wr, We)

    return (ns.reshape(input_shape),
            ti.reshape(*input_shape[:-1], 1),
            mask.reshape(*input_shape[:-1], E),
            rp.reshape(*input_shape[:-1], E),
            probs.reshape(*input_shape[:-1], E))


# speedup vs baseline: 2.9668x; 1.0139x over previous
import jax
import jax.numpy as jnp
from jax.experimental import pallas as pl
from jax.experimental.pallas import tpu as pltpu


def _moe_block_kernel(x_ref, wr_ref, we_ref,
                      ns_ref, ti_ref, mask_ref, rp_ref, probs_ref):
    e = pl.program_id(1)
    num_e = pl.num_programs(1)
    x = x_ref[...]

    @pl.when(e == 0)
    def _router():
        logits = jnp.dot(x, wr_ref[...], preferred_element_type=jnp.float32)
        m = jnp.max(logits, axis=-1, keepdims=True)
        ex = jnp.exp(logits - m)
        probs = ex / jnp.sum(ex, axis=-1, keepdims=True)
        ti = jnp.argmax(probs, axis=-1)
        onehot = (jax.lax.broadcasted_iota(jnp.int32, probs.shape, 1)
                  == ti[:, None]).astype(jnp.float32)
        probs_ref[...] = probs
        mask_ref[...] = onehot
        rp_ref[...] = onehot
        ti_ref[...] = ti[:, None].astype(jnp.int32)
        ns_ref[...] = jnp.zeros_like(ns_ref)

    sel = (ti_ref[...] == e).astype(jnp.float32)
    mm = jnp.dot(x.astype(jnp.bfloat16), we_ref[0].astype(jnp.bfloat16),
                 preferred_element_type=jnp.float32)
    ns_ref[...] += sel * mm


def kernel(x, Wr, We):
    input_shape = x.shape
    D = x.shape[-1]
    E = Wr.shape[-1]
    xf = x.reshape(-1, D)
    N = xf.shape[0]
    T = 2048
    num_t = N // T

    grid = (num_t, E)
    out_shapes = (
        jax.ShapeDtypeStruct((N, D), jnp.float32),
        jax.ShapeDtypeStruct((N, 1), jnp.int32),
        jax.ShapeDtypeStruct((N, E), jnp.float32),
        jax.ShapeDtypeStruct((N, E), jnp.float32),
        jax.ShapeDtypeStruct((N, E), jnp.float32),
    )
    ns, ti, mask, rp, probs = pl.pallas_call(
        _moe_block_kernel,
        grid=grid,
        in_specs=[
            pl.BlockSpec((T, D), lambda t, e: (t, 0)),
            pl.BlockSpec((D, E), lambda t, e: (0, 0)),
            pl.BlockSpec((1, D, D), lambda t, e: (e, 0, 0)),
        ],
        out_specs=(
            pl.BlockSpec((T, D), lambda t, e: (t, 0)),
            pl.BlockSpec((T, 1), lambda t, e: (t, 0)),
            pl.BlockSpec((T, E), lambda t, e: (t, 0)),
            pl.BlockSpec((T, E), lambda t, e: (t, 0)),
            pl.BlockSpec((T, E), lambda t, e: (t, 0)),
        ),
        out_shape=out_shapes,
        compiler_params=pltpu.CompilerParams(
            dimension_semantics=("parallel", "arbitrary"),
        ),
    )(xf, Wr, We)

    return (ns.reshape(input_shape),
            ti.reshape(*input_shape[:-1], 1),
            mask.reshape(*input_shape[:-1], E),
            rp.reshape(*input_shape[:-1], E),
            probs.reshape(*input_shape[:-1], E))
